# baseline (speedup 1.0000x reference)
import functools

import jax
import jax.numpy as jnp
from jax import lax
from jax.experimental import pallas as pl
from jax.experimental.pallas import tpu as pltpu
from jax.experimental.pallas import tpu_sc as plsc

D = 64
NC = 2
NS = 16
NW = NC * NS
L = 16


def _rsqrt_nr(x):
    bits = lax.bitcast_convert_type(x, jnp.int32)
    y = lax.bitcast_convert_type(
        jnp.int32(0x5F3759DF) - lax.shift_right_logical(bits, 1), jnp.float32
    )
    half = x * 0.5
    for _ in range(3):
        y = y * (1.5 - half * y * y)
    return y


def _make_kernel(batch, n_rows):
    b_per_w = batch // NW
    mesh = plsc.VectorSubcoreMesh(core_axis_name="c", subcore_axis_name="s")

    @functools.partial(
        pl.kernel,
        mesh=mesh,
        out_type=jax.ShapeDtypeStruct((batch, D), jnp.float32),
        compiler_params=pltpu.CompilerParams(
            needs_layout_passes=False, use_tc_tiling_on_sc=False
        ),
        scratch_types=[
            pltpu.VMEM((b_per_w,), jnp.int32),
            pltpu.VMEM((b_per_w, D), jnp.float32),
            pltpu.SemaphoreType.DMA,
        ],
    )
    def k(table_hbm, idx_hbm, out_hbm, idx_v, rows_v, sem):
        wid = lax.axis_index("s") * NC + lax.axis_index("c")
        base = wid * b_per_w
        pltpu.sync_copy(idx_hbm.at[pl.ds(base, b_per_w)], idx_v)
        pltpu.async_copy(table_hbm.at[idx_v], rows_v, sem).wait()

        lanes = lax.iota(jnp.int32, L)

        def group_body(g, carry):
            row_ids = lanes + g * L
            acc = None
            for c in range(D):
                col = jnp.full((L,), c, jnp.int32)
                v = plsc.load_gather(rows_v, [row_ids, col])
                acc = v * v if acc is None else acc + v * v
            scale = _rsqrt_nr(acc)
            for j in range(L):
                sc = scale[j]
                i = g * L + j
                for c in range(D // L):
                    rows_v[i, pl.ds(c * L, L)] = rows_v[i, pl.ds(c * L, L)] * sc
            return carry

        lax.fori_loop(0, b_per_w // L, group_body, 0)

        pltpu.sync_copy(rows_v, out_hbm.at[pl.ds(base, b_per_w)])

    return k


def kernel(user_ids, table):
    batch = user_ids.shape[0]
    k = _make_kernel(batch, table.shape[0])
    return k(table, user_ids)


# speedup vs baseline: 1.7169x; 1.7169x over previous
import functools

import jax
import jax.numpy as jnp
from jax import lax
from jax.experimental import pallas as pl
from jax.experimental.pallas import tpu as pltpu
from jax.experimental.pallas import tpu_sc as plsc

D = 64
NC = 2
NS = 16
NW = NC * NS
L = 16


def _rsqrt_nr(x):
    bits = lax.bitcast_convert_type(x, jnp.int32)
    y = lax.bitcast_convert_type(
        jnp.int32(0x5F3759DF) - lax.shift_right_logical(bits, 1), jnp.float32
    )
    half = x * 0.5
    for _ in range(3):
        y = y * (1.5 - half * y * y)
    return y


def _make_kernel(batch):
    b_per_w = batch // NW
    mesh = plsc.VectorSubcoreMesh(core_axis_name="c", subcore_axis_name="s")

    @functools.partial(
        pl.kernel,
        mesh=mesh,
        out_type=jax.ShapeDtypeStruct((batch, D), jnp.float32),
        compiler_params=pltpu.CompilerParams(needs_layout_passes=False),
        scratch_types=[
            pltpu.VMEM((b_per_w,), jnp.int32),
            pltpu.VMEM((b_per_w, D), jnp.float32),
            pltpu.SemaphoreType.DMA,
        ],
    )
    def k(table_hbm, idx_hbm, out_hbm, idx_v, rows_v, sem):
        wid = lax.axis_index("s") * NC + lax.axis_index("c")
        base = wid * b_per_w
        pltpu.sync_copy(idx_hbm.at[pl.ds(base, b_per_w)], idx_v)

        def fire_body(g, carry):
            ids = idx_v[pl.ds(g * L, L)]
            for j in range(L):
                r = ids[j]
                pltpu.async_copy(
                    table_hbm.at[pl.ds(r, 1)],
                    rows_v.at[pl.ds(g * L + j, 1)],
                    sem,
                )
            return carry

        lax.fori_loop(0, b_per_w // L, fire_body, 0)

        def drain_body(i, carry):
            pltpu.make_async_copy(
                table_hbm.at[pl.ds(0, 1)], rows_v.at[pl.ds(0, 1)], sem
            ).wait()
            return carry

        lax.fori_loop(0, b_per_w, drain_body, 0)

        def row_body(i, carry):
            vs = [rows_v[i, pl.ds(c * L, L)] for c in range(D // L)]
            s = vs[0] * vs[0]
            for v in vs[1:]:
                s = s + v * v
            sc = jnp.broadcast_to(_rsqrt_nr(jnp.sum(s)), (L,))
            for c, v in enumerate(vs):
                rows_v[i, pl.ds(c * L, L)] = v * sc
            return carry

        lax.fori_loop(0, b_per_w, row_body, 0, unroll=2)

        pltpu.sync_copy(rows_v, out_hbm.at[pl.ds(base, b_per_w)])

    return k


def kernel(user_ids, table):
    batch = user_ids.shape[0]
    k = _make_kernel(batch)
    return k(table, user_ids)
